# Initial kernel scaffold; baseline (speedup 1.0000x reference)
import jax
import jax.numpy as jnp
from jax.experimental import pallas as pl


def kernel(input_ids, table, pos_emb):
    raise NotImplementedError("write your pallas kernel here")


# speedup vs baseline: 3.7769x; 3.7769x over previous
import functools

import jax
import jax.numpy as jnp
import numpy as np
from jax import lax
from jax.experimental import pallas as pl
from jax.experimental.pallas import tpu as pltpu
from jax.experimental.pallas import tpu_sc as plsc

VOCAB = 100000
HIDDEN = 768
SEQ = 2048
BATCH = 4
SCALE = float(np.sqrt(HIDDEN))

N = BATCH * SEQ
NW = 32
PER_W = N // NW
CHUNK = 32
NCHUNK = PER_W // CHUNK
NV = HIDDEN // 16


def _sc_embed(ids, table, pos_emb):
    mesh = plsc.VectorSubcoreMesh(core_axis_name="c", subcore_axis_name="s")

    @functools.partial(
        pl.kernel,
        out_type=jax.ShapeDtypeStruct((N, HIDDEN), jnp.float32),
        mesh=mesh,
        scratch_types=[
            pltpu.VMEM((PER_W,), jnp.int32),
            pltpu.VMEM((CHUNK, HIDDEN), jnp.float32),
            pltpu.VMEM((CHUNK, HIDDEN), jnp.float32),
            pltpu.SemaphoreType.DMA,
        ],
    )
    def k(ids_hbm, table_hbm, pos_hbm, out_hbm, idx_v, rows_v, pos_v, sem):
        wid = lax.axis_index("s") * 2 + lax.axis_index("c")
        base = wid * PER_W
        pos_base = lax.rem(base, SEQ)

        pltpu.sync_copy(ids_hbm.at[pl.ds(base, PER_W)], idx_v)

        def chunk_body(t, _):
            off = t * CHUNK
            gather = pltpu.async_copy(
                table_hbm.at[idx_v.at[pl.ds(off, CHUNK)]], rows_v, sem
            )
            pltpu.sync_copy(pos_hbm.at[pl.ds(pos_base + off, CHUNK)], pos_v)
            gather.wait()

            def row_body(i, _):
                for j in range(NV):
                    sl = pl.ds(j * 16, 16)
                    rows_v[i, sl] = rows_v[i, sl] * SCALE + pos_v[i, sl]
                return 0

            lax.fori_loop(0, CHUNK, row_body, 0)
            pltpu.sync_copy(rows_v, out_hbm.at[pl.ds(base + off, CHUNK)])
            return 0

        lax.fori_loop(0, NCHUNK, chunk_body, 0)

    return k(ids, table, pos_emb)


def kernel(input_ids, table, pos_emb):
    ids = input_ids.reshape(-1).astype(jnp.int32)
    out = _sc_embed(ids, table, pos_emb)
    return out.reshape(BATCH, SEQ, HIDDEN)


# speedup vs baseline: 4.6634x; 1.2347x over previous
import functools

import jax
import jax.numpy as jnp
import numpy as np
from jax import lax
from jax.experimental import pallas as pl
from jax.experimental.pallas import tpu as pltpu
from jax.experimental.pallas import tpu_sc as plsc

VOCAB = 100000
HIDDEN = 768
SEQ = 2048
BATCH = 4
SCALE = float(np.sqrt(HIDDEN))

N = BATCH * SEQ
NW = 32
PER_W = N // NW
CHUNK = 16
NCHUNK = PER_W // CHUNK
NV = HIDDEN // 16
NBUF = 2


def _sc_embed(ids, table, pos_emb):
    mesh = plsc.VectorSubcoreMesh(core_axis_name="c", subcore_axis_name="s")

    @functools.partial(
        pl.kernel,
        out_type=jax.ShapeDtypeStruct((N, HIDDEN), jnp.float32),
        mesh=mesh,
        scratch_types=[
            pltpu.VMEM((PER_W,), jnp.int32),
            pltpu.VMEM((NBUF, CHUNK, HIDDEN), jnp.float32),
            pltpu.VMEM((NBUF, CHUNK, HIDDEN), jnp.float32),
            pltpu.VMEM((NBUF, CHUNK, HIDDEN), jnp.float32),
            pltpu.SemaphoreType.DMA,
            pltpu.SemaphoreType.DMA,
            pltpu.SemaphoreType.DMA,
        ],
    )
    def k(ids_hbm, table_hbm, pos_hbm, out_hbm, idx_v, rows_v, pos_v, res_v,
          sem_g, sem_p, sem_o):
        wid = lax.axis_index("s") * 2 + lax.axis_index("c")
        base = wid * PER_W
        pos_base = lax.rem(base, SEQ)

        pltpu.sync_copy(ids_hbm.at[pl.ds(base, PER_W)], idx_v)

        def issue_in(t, b):
            off = t * CHUNK
            pltpu.async_copy(
                table_hbm.at[idx_v.at[pl.ds(off, CHUNK)]], rows_v.at[b], sem_g
            )
            pltpu.async_copy(
                pos_hbm.at[pl.ds(pos_base + off, CHUNK)], pos_v.at[b], sem_p
            )

        for b in range(NBUF):
            issue_in(b, b)

        def outer(g, _):
            for b in range(NBUF):
                t = NBUF * g + b
                pltpu.make_async_copy(
                    table_hbm.at[pl.ds(0, CHUNK)], rows_v.at[b], sem_g
                ).wait()
                pltpu.make_async_copy(
                    pos_hbm.at[pl.ds(0, CHUNK)], pos_v.at[b], sem_p
                ).wait()
                @pl.when(t >= NBUF)
                def _():
                    pltpu.make_async_copy(
                        res_v.at[b], out_hbm.at[pl.ds(0, CHUNK)], sem_o
                    ).wait()

                def row_body(i, _):
                    for j in range(NV):
                        sl = pl.ds(j * 16, 16)
                        res_v[b, i, sl] = (
                            rows_v[b, i, sl] * SCALE + pos_v[b, i, sl]
                        )
                    return 0

                lax.fori_loop(0, CHUNK, row_body, 0)

                pltpu.async_copy(
                    res_v.at[b], out_hbm.at[pl.ds(base + t * CHUNK, CHUNK)],
                    sem_o,
                )

                @pl.when(t + NBUF < NCHUNK)
                def _():
                    issue_in(t + NBUF, b)

            return 0

        lax.fori_loop(0, NCHUNK // NBUF, outer, 0)

        for b in range(NBUF):
            pltpu.make_async_copy(
                res_v.at[b], out_hbm.at[pl.ds(0, CHUNK)], sem_o
            ).wait()

    return k(ids, table, pos_emb)


def kernel(input_ids, table, pos_emb):
    ids = input_ids.reshape(-1).astype(jnp.int32)
    out = _sc_embed(ids, table, pos_emb)
    return out.reshape(BATCH, SEQ, HIDDEN)
